# baseline (speedup 1.0000x reference)
import functools
import math

import jax
import jax.numpy as jnp
import numpy as np
from jax import lax
from jax.experimental import pallas as pl
from jax.experimental.pallas import tpu as pltpu
from jax.experimental.pallas import tpu_sc as plsc

B, N, D = 2, 2048, 1024
E, TOPK, DB = 8, 2, 128
CAP = 1280
T = B * N
TK = T * TOPK
EP = 128
NEG = -1e30

L = 16
NS = 16
EC = E * CAP
PAD_ROWS = EC + L
MARK = EC + (1 << 20)
TPS = T // NS


def _router_body(x_ref, wr_ref, idx_ref, rp_ref, stats_ref,
                 e0_ref, e1_ref, rp0_ref, rp1_ref):
    x = x_ref[...]
    wr = wr_ref[...]
    logits = jnp.dot(x, wr, preferred_element_type=jnp.float32)
    cols = jax.lax.broadcasted_iota(jnp.int32, (T, EP), 1)
    valid = cols < E
    logits = jnp.where(valid, logits, NEG)
    m = jnp.max(logits, axis=1, keepdims=True)
    p = jnp.exp(logits - m)
    probs = p / jnp.sum(p, axis=1, keepdims=True)
    m1 = jnp.max(probs, axis=1, keepdims=True)
    i1 = jnp.min(jnp.where(probs == m1, cols, EP), axis=1, keepdims=True)
    probs2 = jnp.where(cols == i1, -1.0, probs)
    m2 = jnp.max(probs2, axis=1, keepdims=True)
    i2 = jnp.min(jnp.where(probs2 == m2, cols, EP), axis=1, keepdims=True)
    ssum = m1 + m2 + 1e-9
    rp1 = m1 / ssum
    rp2 = m2 / ssum
    idx_ref[...] = jnp.concatenate([i1, i2], axis=1)
    rp_ref[...] = jnp.concatenate([rp1, rp2], axis=1)
    e0_ref[...] = i1
    e1_ref[...] = i2
    rp0_ref[...] = rp1
    rp1_ref[...] = rp2
    onehot = (cols == i1).astype(jnp.float32) + (cols == i2).astype(jnp.float32)
    counts = jnp.sum(onehot, axis=0, keepdims=True)
    psum = jnp.sum(probs, axis=0, keepdims=True)
    denom = float(T) + 1e-9
    f_e = counts / (TOPK * denom)
    p_e = psum / denom
    loss = E * jnp.sum(f_e * p_e, axis=1, keepdims=True)
    maxvio = jnp.max(f_e, axis=1, keepdims=True) * E - 1.0
    stats_ref[...] = jnp.concatenate(
        [jnp.concatenate([loss, maxvio], axis=1), counts[:, : EP - 2]], axis=1)


def _router(x, wr_pad):
    return pl.pallas_call(
        _router_body,
        out_shape=(
            jax.ShapeDtypeStruct((T, 2), jnp.int32),
            jax.ShapeDtypeStruct((T, 2), jnp.float32),
            jax.ShapeDtypeStruct((1, EP), jnp.float32),
            jax.ShapeDtypeStruct((T, 1), jnp.int32),
            jax.ShapeDtypeStruct((T, 1), jnp.int32),
            jax.ShapeDtypeStruct((T, 1), jnp.float32),
            jax.ShapeDtypeStruct((T, 1), jnp.float32),
        ),
    )(x, wr_pad)


NW = 32
CW = 32
TPB = T // NW
CH = TPB // CW
TKW = 128


def _sortpack_body(e0_hbm, e1_hbm, x_hbm,
                   d0_hbm, d1_hbm, tok2d_hbm, packed_hbm,
                   e0_v, e1_v, p0_v, p1_v,
                   tmp_v, comb0_v, comb1_v, e0r_v, e1r_v,
                   d0o_v, d1o_v, d0c_v, d1c_v, rows_tok, xbuf0, xbuf1,
                   sh_p0, sh_p1, sem, sem2, sem3):
    cid = lax.axis_index("c")
    sid = lax.axis_index("s")
    wid = sid * 2 + cid

    def eq1(v, s):
        return 1 - jnp.minimum(jnp.abs(v - s), 1)

    @pl.when(sid < E)
    def _scan():
        e = sid
        pltpu.sync_copy(e0_hbm, e0_v)
        pltpu.sync_copy(e1_hbm, e1_v)
        esplat = jnp.full((L,), e, jnp.int32)

        def step(i, carry):
            sl = pl.ds(i * L, L)
            m0 = eq1(e0_v[sl], esplat)
            m1 = eq1(e1_v[sl], esplat)
            c0 = plsc.cumsum(m0)
            c1 = plsc.cumsum(m1)
            i1 = c1 - m1
            pos0 = carry + (c0 - m0) + i1
            pos1 = carry + c0 + i1
            p0_v[sl] = m0 * (pos0 + 1)
            p1_v[sl] = m1 * (pos1 + 1)
            s = jnp.sum(m0) + jnp.sum(m1)
            return carry + s

        lax.fori_loop(0, T // L, step, jnp.zeros((L,), jnp.int32))
        pltpu.sync_copy(p0_v, sh_p0.at[e])
        pltpu.sync_copy(p1_v, sh_p1.at[e])

    plsc.subcore_barrier()

    tb = wid * TPB
    pltpu.sync_copy(e0_hbm.at[pl.ds(tb, TPB)], e0r_v)
    pltpu.sync_copy(e1_hbm.at[pl.ds(tb, TPB)], e1r_v)
    pltpu.sync_copy(sh_p0.at[0, pl.ds(tb, TPB)], comb0_v)
    pltpu.sync_copy(sh_p1.at[0, pl.ds(tb, TPB)], comb1_v)
    for e in range(1, E):
        pltpu.sync_copy(sh_p0.at[e, pl.ds(tb, TPB)], tmp_v)
        for j in range(TPB // L):
            sl = pl.ds(j * L, L)
            comb0_v[sl] = comb0_v[sl] + tmp_v[sl]
        pltpu.sync_copy(sh_p1.at[e, pl.ds(tb, TPB)], tmp_v)
        for j in range(TPB // L):
            sl = pl.ds(j * L, L)
            comb1_v[sl] = comb1_v[sl] + tmp_v[sl]
    for j in range(TPB // L):
        sl = pl.ds(j * L, L)
        p0 = comb0_v[sl] - 1
        p1 = comb1_v[sl] - 1
        ge0 = jnp.minimum(jnp.maximum(p0 - (CAP - 1), 0), 1)
        ge1 = jnp.minimum(jnp.maximum(p1 - (CAP - 1), 0), 1)
        dd0 = e0r_v[sl] * CAP + p0
        dd1 = e1r_v[sl] * CAP + p1
        d0o_v[sl] = dd0 + ge0 * (MARK - dd0)
        d1o_v[sl] = dd1 + ge1 * (MARK - dd1)
        csl = pl.ds((j % (CW // L)) * L, L)
        d0c_v[j // (CW // L), csl] = dd0 + ge0 * (EC - dd0)
        d1c_v[j // (CW // L), csl] = dd1 + ge1 * (EC - dd1)
    pltpu.sync_copy(d0o_v, d0_hbm.at[pl.ds(tb, TPB)])
    pltpu.sync_copy(d1o_v, d1_hbm.at[pl.ds(tb, TPB)])

    bufs = (xbuf0, xbuf1)
    lds = [None] * CH
    for c in range(min(2, CH)):
        lds[c] = pltpu.async_copy(
            x_hbm.at[pl.ds(tb + c * CW, CW)], bufs[c % 2], sem3)
    for c in range(CH):
        cbase = tb + c * CW

        def rstep(j, acc):
            for kk in range(TKW // L):
                rows_tok[j, pl.ds(kk * L, L)] = jnp.full((L,), cbase + j,
                                                         jnp.int32)
            return acc

        lax.fori_loop(0, CW, rstep, 0)
        t0 = pltpu.async_copy(rows_tok, tok2d_hbm.at[d0c_v.at[c]], sem)
        t1 = pltpu.async_copy(rows_tok, tok2d_hbm.at[d1c_v.at[c]], sem2)
        lds[c].wait()
        s0 = pltpu.async_copy(bufs[c % 2], packed_hbm.at[d0c_v.at[c]], sem)
        s1 = pltpu.async_copy(bufs[c % 2], packed_hbm.at[d1c_v.at[c]], sem2)
        t0.wait()
        t1.wait()
        s0.wait()
        s1.wait()
        if c + 2 < CH:
            lds[c + 2] = pltpu.async_copy(
                x_hbm.at[pl.ds(tb + (c + 2) * CW, CW)], bufs[c % 2], sem3)


def _sortpack(e0, e1, x):
    mesh = plsc.VectorSubcoreMesh(core_axis_name="c", subcore_axis_name="s")
    f = pl.kernel(
        _sortpack_body,
        out_type=(
            jax.ShapeDtypeStruct((T,), jnp.int32),
            jax.ShapeDtypeStruct((T,), jnp.int32),
            jax.ShapeDtypeStruct((PAD_ROWS, TKW), jnp.int32),
            jax.ShapeDtypeStruct((PAD_ROWS, D), jnp.float32),
        ),
        mesh=mesh,
        compiler_params=pltpu.CompilerParams(needs_layout_passes=False),
        scratch_types=[
            pltpu.VMEM((T,), jnp.int32),
            pltpu.VMEM((T,), jnp.int32),
            pltpu.VMEM((T,), jnp.int32),
            pltpu.VMEM((T,), jnp.int32),
            pltpu.VMEM((TPB,), jnp.int32),
            pltpu.VMEM((TPB,), jnp.int32),
            pltpu.VMEM((TPB,), jnp.int32),
            pltpu.VMEM((TPB,), jnp.int32),
            pltpu.VMEM((TPB,), jnp.int32),
            pltpu.VMEM((TPB,), jnp.int32),
            pltpu.VMEM((TPB,), jnp.int32),
            pltpu.VMEM((CH, CW), jnp.int32),
            pltpu.VMEM((CH, CW), jnp.int32),
            pltpu.VMEM((CW, TKW), jnp.int32),
            pltpu.VMEM((CW, D), jnp.float32),
            pltpu.VMEM((CW, D), jnp.float32),
            pltpu.VMEM_SHARED((E, T), jnp.int32),
            pltpu.VMEM_SHARED((E, T), jnp.int32),
            pltpu.SemaphoreType.DMA,
            pltpu.SemaphoreType.DMA,
            pltpu.SemaphoreType.DMA,
        ],
    )
    return f(e0, e1, x)


UW = 16
UCH = TPB // UW


def _unpack_body(po_hbm, d0_hbm, d1_hbm, rp0_hbm, rp1_hbm, out_hbm,
                 d0_v, d1_v, d0s_v, d1s_v, rp0_v, rp1_v,
                 buf0a, buf0b, buf1a, buf1b, obufa, obufb,
                 g0s, g1s, oss):
    wid = lax.axis_index("s") * 2 + lax.axis_index("c")
    cb = wid * UCH
    tb = wid * TPB
    pltpu.sync_copy(d0_hbm.at[pl.ds(cb, UCH)], d0_v)
    pltpu.sync_copy(d1_hbm.at[pl.ds(cb, UCH)], d1_v)
    pltpu.sync_copy(rp0_hbm.at[pl.ds(cb, UCH)], rp0_v)
    pltpu.sync_copy(rp1_hbm.at[pl.ds(cb, UCH)], rp1_v)
    for c in range(UCH):
        v0 = d0_v[c, :]
        v1 = d1_v[c, :]
        g0 = jnp.minimum(jnp.maximum(v0 - (EC - 1), 0), 1)
        g1 = jnp.minimum(jnp.maximum(v1 - (EC - 1), 0), 1)
        d0s_v[c, :] = v0 - g0 * v0
        d1s_v[c, :] = v1 - g1 * v1
    b0 = (buf0a, buf0b)
    b1 = (buf1a, buf1b)
    ob = (obufa, obufb)
    cp0 = [None] * UCH
    cp1 = [None] * UCH
    ost = [None, None]
    for c in range(min(2, UCH)):
        cp0[c] = pltpu.async_copy(po_hbm.at[d0s_v.at[c]], b0[c % 2], g0s)
        cp1[c] = pltpu.async_copy(po_hbm.at[d1s_v.at[c]], b1[c % 2], g1s)
    for c in range(UCH):
        k = c % 2
        cp0[c].wait()
        cp1[c].wait()
        if ost[k] is not None:
            ost[k].wait()

        def rstep(r, acc):
            lm = 1 - jnp.minimum(
                jnp.abs(lax.iota(jnp.int32, L) - r), 1)
            lmf = lm.astype(jnp.float32)
            raw0 = jnp.sum(d0_v[c, :] * lm)
            raw1 = jnp.sum(d1_v[c, :] * lm)
            ge0 = jnp.minimum(jnp.maximum(raw0 - (EC - 1), 0), 1)
            ge1 = jnp.minimum(jnp.maximum(raw1 - (EC - 1), 0), 1)
            s0 = jnp.sum(rp0_v[c, :] * lmf) * (1 - ge0).astype(jnp.float32)
            s1 = jnp.sum(rp1_v[c, :] * lmf) * (1 - ge1).astype(jnp.float32)
            s0v = jnp.full((L,), s0, jnp.float32)
            s1v = jnp.full((L,), s1, jnp.float32)
            for kk in range(D // L):
                sl = pl.ds(kk * L, L)
                ob[k][r, sl] = s0v * b0[k][r, sl] + s1v * b1[k][r, sl]
            return acc

        lax.fori_loop(0, UW, rstep, 0)
        if c + 2 < UCH:
            cp0[c + 2] = pltpu.async_copy(po_hbm.at[d0s_v.at[c + 2]],
                                          b0[k], g0s)
            cp1[c + 2] = pltpu.async_copy(po_hbm.at[d1s_v.at[c + 2]],
                                          b1[k], g1s)
        ost[k] = pltpu.async_copy(ob[k], out_hbm.at[pl.ds(tb + c * UW, UW)],
                                  oss)
    for k in range(2):
        if ost[k] is not None:
            ost[k].wait()


def _unpack(po, d0r, d1r, rp0r, rp1r):
    mesh = plsc.VectorSubcoreMesh(core_axis_name="c", subcore_axis_name="s")
    f = pl.kernel(
        _unpack_body,
        out_type=jax.ShapeDtypeStruct((T, D), jnp.float32),
        mesh=mesh,
        compiler_params=pltpu.CompilerParams(needs_layout_passes=False),
        scratch_types=[
            pltpu.VMEM((UCH, UW), jnp.int32),
            pltpu.VMEM((UCH, UW), jnp.int32),
            pltpu.VMEM((UCH, UW), jnp.int32),
            pltpu.VMEM((UCH, UW), jnp.int32),
            pltpu.VMEM((UCH, UW), jnp.float32),
            pltpu.VMEM((UCH, UW), jnp.float32),
            pltpu.VMEM((UW, D), jnp.float32),
            pltpu.VMEM((UW, D), jnp.float32),
            pltpu.VMEM((UW, D), jnp.float32),
            pltpu.VMEM((UW, D), jnp.float32),
            pltpu.VMEM((UW, D), jnp.float32),
            pltpu.VMEM((UW, D), jnp.float32),
            pltpu.SemaphoreType.DMA,
            pltpu.SemaphoreType.DMA,
            pltpu.SemaphoreType.DMA,
        ],
    )
    return f(po, d0r, d1r, rp0r, rp1r)


BQ = 256
NQB = CAP // BQ


def _attn_body(cnt_ref, px_ref, tok_ref, wq_ref, wk_ref, wv_ref, wo_ref,
               out_ref, k_scr, v_scr):
    cnt = cnt_ref[0, 0, 0]
    qi = pl.program_id(1)
    qbase = qi * BQ

    @pl.when(qi == 0)
    def _kv():
        x = px_ref[...]
        ri = jax.lax.broadcasted_iota(jnp.int32, (CAP, 1), 0)
        xm = jnp.where(ri < cnt, x, 0.0)
        k_scr[...] = jnp.dot(xm, wk_ref[0], preferred_element_type=jnp.float32)
        v_scr[...] = jnp.dot(xm, wv_ref[0], preferred_element_type=jnp.float32)

    @pl.when(qbase < cnt)
    def _compute():
        ri = jax.lax.broadcasted_iota(jnp.int32, (BQ, 1), 0) + qbase
        act = (ri < cnt).astype(jnp.float32)
        xq = jnp.where(ri < cnt, px_ref[pl.ds(qbase, BQ), :], 0.0)
        q = jnp.dot(xq, wq_ref[0],
                    preferred_element_type=jnp.float32) * (1.0 / math.sqrt(DB))
        scores = jax.lax.dot_general(
            q, k_scr[...], (((1,), (1,)), ((), ())),
            preferred_element_type=jnp.float32)
        tok = tok_ref[0]
        pos_k = tok % N
        b_k = tok // N
        tokq = tok_ref[0, :, pl.ds(qbase, BQ)]
        pos_q = jnp.reshape(tokq % N, (BQ, 1))
        b_q = jnp.reshape(tokq // N, (BQ, 1))
        ci = jax.lax.broadcasted_iota(jnp.int32, (1, CAP), 1)
        mask = (ci < cnt) & (b_q == b_k) & (pos_k <= pos_q)
        scores = jnp.where(mask, scores, -1e9)
        m = jnp.max(scores, axis=1, keepdims=True)
        pexp = jnp.exp(scores - m)
        ctx = jnp.dot(pexp, v_scr[...], preferred_element_type=jnp.float32)
        ctx = ctx / jnp.sum(pexp, axis=1, keepdims=True)
        out_ref[...] = jnp.dot(
            ctx, wo_ref[0], preferred_element_type=jnp.float32) * act

    @pl.when(qbase >= cnt)
    def _skip():
        out_ref[...] = jnp.zeros((BQ, D), jnp.float32)


def _attention(counts, packed_x, slot_tok, Wq, Wk, Wv, Wo):
    grid = (E, NQB)
    return pl.pallas_call(
        _attn_body,
        grid=grid,
        in_specs=[
            pl.BlockSpec((1, 1, 1), lambda e, qi: (e, 0, 0)),
            pl.BlockSpec((CAP, D), lambda e, qi: (e, 0)),
            pl.BlockSpec((1, 1, CAP), lambda e, qi: (e, 0, 0)),
            pl.BlockSpec((1, D, DB), lambda e, qi: (e, 0, 0)),
            pl.BlockSpec((1, D, DB), lambda e, qi: (e, 0, 0)),
            pl.BlockSpec((1, D, DB), lambda e, qi: (e, 0, 0)),
            pl.BlockSpec((1, DB, D), lambda e, qi: (e, 0, 0)),
        ],
        out_specs=pl.BlockSpec((BQ, D), lambda e, qi: (e * NQB + qi, 0)),
        out_shape=jax.ShapeDtypeStruct((EC, D), jnp.float32),
        scratch_shapes=[
            pltpu.VMEM((CAP, DB), jnp.float32),
            pltpu.VMEM((CAP, DB), jnp.float32),
        ],
        compiler_params=pltpu.CompilerParams(
            dimension_semantics=("arbitrary", "arbitrary")),
    )(counts, packed_x, slot_tok, Wq, Wk, Wv, Wo)


def kernel(hidden_states, position_ids, active_mask, W_router, Wq, Wk, Wv, Wo):
    x = hidden_states.reshape(T, D)
    wr_pad = jnp.pad(W_router, ((0, 0), (0, EP - E)))
    idx, rp, stats, e0c, e1c, rp0c, rp1c = _router(x, wr_pad)
    loss = stats[0, 0]
    maxvio = stats[0, 1]
    d0, d1, tok2d, packed = _sortpack(e0c.reshape(T), e1c.reshape(T), x)
    counts = stats[0, 2:2 + E].astype(jnp.int32)

    slot_tok = tok2d[:EC, 0].reshape(E, 1, CAP)
    po = _attention(counts.reshape(E, 1, 1), packed,
                    slot_tok, Wq, Wk, Wv, Wo)

    rp0r = rp0c.reshape(NW * UCH, UW)
    rp1r = rp1c.reshape(NW * UCH, UW)
    final = _unpack(po, d0.reshape(NW * UCH, UW), d1.reshape(NW * UCH, UW),
                    rp0r, rp1r).reshape(B, N, D)
    return final, loss, maxvio


# speedup vs baseline: 1.2375x; 1.2375x over previous
import functools
import math

import jax
import jax.numpy as jnp
import numpy as np
from jax import lax
from jax.experimental import pallas as pl
from jax.experimental.pallas import tpu as pltpu
from jax.experimental.pallas import tpu_sc as plsc

B, N, D = 2, 2048, 1024
E, TOPK, DB = 8, 2, 128
CAP = 1280
T = B * N
TK = T * TOPK
EP = 128
NEG = -1e30

L = 16
NS = 16
EC = E * CAP
PAD_ROWS = EC + L
MARK = EC + (1 << 20)
TPS = T // NS


def _router_body(x_ref, wr_ref, idx_ref, rp_ref, stats_ref,
                 e0_ref, e1_ref, rp0_ref, rp1_ref):
    x = x_ref[...]
    wr = wr_ref[...]
    logits = jnp.dot(x, wr, preferred_element_type=jnp.float32)
    cols = jax.lax.broadcasted_iota(jnp.int32, (T, EP), 1)
    valid = cols < E
    logits = jnp.where(valid, logits, NEG)
    m = jnp.max(logits, axis=1, keepdims=True)
    p = jnp.exp(logits - m)
    probs = p / jnp.sum(p, axis=1, keepdims=True)
    m1 = jnp.max(probs, axis=1, keepdims=True)
    i1 = jnp.min(jnp.where(probs == m1, cols, EP), axis=1, keepdims=True)
    probs2 = jnp.where(cols == i1, -1.0, probs)
    m2 = jnp.max(probs2, axis=1, keepdims=True)
    i2 = jnp.min(jnp.where(probs2 == m2, cols, EP), axis=1, keepdims=True)
    ssum = m1 + m2 + 1e-9
    rp1 = m1 / ssum
    rp2 = m2 / ssum
    idx_ref[...] = jnp.concatenate([i1, i2], axis=1)
    rp_ref[...] = jnp.concatenate([rp1, rp2], axis=1)
    e0_ref[...] = i1
    e1_ref[...] = i2
    rp0_ref[...] = rp1
    rp1_ref[...] = rp2
    onehot = (cols == i1).astype(jnp.float32) + (cols == i2).astype(jnp.float32)
    counts = jnp.sum(onehot, axis=0, keepdims=True)
    psum = jnp.sum(probs, axis=0, keepdims=True)
    denom = float(T) + 1e-9
    f_e = counts / (TOPK * denom)
    p_e = psum / denom
    loss = E * jnp.sum(f_e * p_e, axis=1, keepdims=True)
    maxvio = jnp.max(f_e, axis=1, keepdims=True) * E - 1.0
    stats_ref[...] = jnp.concatenate(
        [jnp.concatenate([loss, maxvio], axis=1), counts[:, : EP - 2]], axis=1)


def _router(x, wr_pad):
    return pl.pallas_call(
        _router_body,
        out_shape=(
            jax.ShapeDtypeStruct((T, 2), jnp.int32),
            jax.ShapeDtypeStruct((T, 2), jnp.float32),
            jax.ShapeDtypeStruct((1, EP), jnp.float32),
            jax.ShapeDtypeStruct((T, 1), jnp.int32),
            jax.ShapeDtypeStruct((T, 1), jnp.int32),
            jax.ShapeDtypeStruct((T, 1), jnp.float32),
            jax.ShapeDtypeStruct((T, 1), jnp.float32),
        ),
    )(x, wr_pad)


NW = 32
CW = 32
TPB = T // NW
CH = TPB // CW
TKW = 128


def _sortpack_body(e0_hbm, e1_hbm, x_hbm,
                   d0_hbm, d1_hbm, tok2d_hbm, packed_hbm,
                   e0_v, e1_v, p0_v, p1_v,
                   tmp_v, comb0_v, comb1_v, e0r_v, e1r_v,
                   d0o_v, d1o_v, d0c_v, d1c_v, rows_tok, xbuf0, xbuf1,
                   sh_p0, sh_p1, sem, sem2, sem3):
    cid = lax.axis_index("c")
    sid = lax.axis_index("s")
    wid = sid * 2 + cid

    def eq1(v, s):
        return 1 - jnp.minimum(jnp.abs(v - s), 1)

    @pl.when(sid < E)
    def _scan():
        e = sid
        pltpu.sync_copy(e0_hbm, e0_v)
        pltpu.sync_copy(e1_hbm, e1_v)
        esplat = jnp.full((L,), e, jnp.int32)

        def step(i, carry):
            sl = pl.ds(i * L, L)
            m0 = eq1(e0_v[sl], esplat)
            m1 = eq1(e1_v[sl], esplat)
            c0 = plsc.cumsum(m0)
            c1 = plsc.cumsum(m1)
            i1 = c1 - m1
            pos0 = carry + (c0 - m0) + i1
            pos1 = carry + c0 + i1
            p0_v[sl] = m0 * (pos0 + 1)
            p1_v[sl] = m1 * (pos1 + 1)
            s = jnp.sum(m0) + jnp.sum(m1)
            return carry + s

        lax.fori_loop(0, T // L, step, jnp.zeros((L,), jnp.int32))
        pltpu.sync_copy(p0_v, sh_p0.at[e])
        pltpu.sync_copy(p1_v, sh_p1.at[e])

    plsc.subcore_barrier()

    tb = wid * TPB
    pltpu.sync_copy(e0_hbm.at[pl.ds(tb, TPB)], e0r_v)
    pltpu.sync_copy(e1_hbm.at[pl.ds(tb, TPB)], e1r_v)
    pltpu.sync_copy(sh_p0.at[0, pl.ds(tb, TPB)], comb0_v)
    pltpu.sync_copy(sh_p1.at[0, pl.ds(tb, TPB)], comb1_v)
    for e in range(1, E):
        pltpu.sync_copy(sh_p0.at[e, pl.ds(tb, TPB)], tmp_v)
        for j in range(TPB // L):
            sl = pl.ds(j * L, L)
            comb0_v[sl] = comb0_v[sl] + tmp_v[sl]
        pltpu.sync_copy(sh_p1.at[e, pl.ds(tb, TPB)], tmp_v)
        for j in range(TPB // L):
            sl = pl.ds(j * L, L)
            comb1_v[sl] = comb1_v[sl] + tmp_v[sl]
    for j in range(TPB // L):
        sl = pl.ds(j * L, L)
        p0 = comb0_v[sl] - 1
        p1 = comb1_v[sl] - 1
        ge0 = jnp.minimum(jnp.maximum(p0 - (CAP - 1), 0), 1)
        ge1 = jnp.minimum(jnp.maximum(p1 - (CAP - 1), 0), 1)
        dd0 = e0r_v[sl] * CAP + p0
        dd1 = e1r_v[sl] * CAP + p1
        d0o_v[sl] = dd0 + ge0 * (MARK - dd0)
        d1o_v[sl] = dd1 + ge1 * (MARK - dd1)
        csl = pl.ds((j % (CW // L)) * L, L)
        d0c_v[j // (CW // L), csl] = dd0 + ge0 * (EC - dd0)
        d1c_v[j // (CW // L), csl] = dd1 + ge1 * (EC - dd1)
    pltpu.sync_copy(d0o_v, d0_hbm.at[pl.ds(tb, TPB)])
    pltpu.sync_copy(d1o_v, d1_hbm.at[pl.ds(tb, TPB)])

    bufs = (xbuf0, xbuf1)
    lds = [None] * CH
    for c in range(min(2, CH)):
        lds[c] = pltpu.async_copy(
            x_hbm.at[pl.ds(tb + c * CW, CW)], bufs[c % 2], sem3)
    for c in range(CH):
        cbase = tb + c * CW

        def rstep(j, acc):
            for kk in range(TKW // L):
                rows_tok[j, pl.ds(kk * L, L)] = jnp.full((L,), cbase + j,
                                                         jnp.int32)
            return acc

        lax.fori_loop(0, CW, rstep, 0)
        t0 = pltpu.async_copy(rows_tok, tok2d_hbm.at[d0c_v.at[c]], sem)
        t1 = pltpu.async_copy(rows_tok, tok2d_hbm.at[d1c_v.at[c]], sem2)
        lds[c].wait()
        s0 = pltpu.async_copy(bufs[c % 2], packed_hbm.at[d0c_v.at[c]], sem)
        s1 = pltpu.async_copy(bufs[c % 2], packed_hbm.at[d1c_v.at[c]], sem2)
        t0.wait()
        t1.wait()
        s0.wait()
        s1.wait()
        if c + 2 < CH:
            lds[c + 2] = pltpu.async_copy(
                x_hbm.at[pl.ds(tb + (c + 2) * CW, CW)], bufs[c % 2], sem3)


def _sortpack(e0, e1, x):
    mesh = plsc.VectorSubcoreMesh(core_axis_name="c", subcore_axis_name="s")
    f = pl.kernel(
        _sortpack_body,
        out_type=(
            jax.ShapeDtypeStruct((T,), jnp.int32),
            jax.ShapeDtypeStruct((T,), jnp.int32),
            jax.ShapeDtypeStruct((PAD_ROWS, TKW), jnp.int32),
            jax.ShapeDtypeStruct((PAD_ROWS, D), jnp.float32),
        ),
        mesh=mesh,
        compiler_params=pltpu.CompilerParams(needs_layout_passes=False),
        scratch_types=[
            pltpu.VMEM((T,), jnp.int32),
            pltpu.VMEM((T,), jnp.int32),
            pltpu.VMEM((T,), jnp.int32),
            pltpu.VMEM((T,), jnp.int32),
            pltpu.VMEM((TPB,), jnp.int32),
            pltpu.VMEM((TPB,), jnp.int32),
            pltpu.VMEM((TPB,), jnp.int32),
            pltpu.VMEM((TPB,), jnp.int32),
            pltpu.VMEM((TPB,), jnp.int32),
            pltpu.VMEM((TPB,), jnp.int32),
            pltpu.VMEM((TPB,), jnp.int32),
            pltpu.VMEM((CH, CW), jnp.int32),
            pltpu.VMEM((CH, CW), jnp.int32),
            pltpu.VMEM((CW, TKW), jnp.int32),
            pltpu.VMEM((CW, D), jnp.float32),
            pltpu.VMEM((CW, D), jnp.float32),
            pltpu.VMEM_SHARED((E, T), jnp.int32),
            pltpu.VMEM_SHARED((E, T), jnp.int32),
            pltpu.SemaphoreType.DMA,
            pltpu.SemaphoreType.DMA,
            pltpu.SemaphoreType.DMA,
        ],
    )
    return f(e0, e1, x)


UW = 16
UCH = TPB // UW


def _unpack_body(po_hbm, d0_hbm, d1_hbm, rp0_hbm, rp1_hbm, out_hbm,
                 d0_v, d1_v, d0s_v, d1s_v, rp0_v, rp1_v,
                 buf0a, buf0b, buf1a, buf1b, obufa, obufb,
                 g0s, g1s, oss):
    wid = lax.axis_index("s") * 2 + lax.axis_index("c")
    cb = wid * UCH
    tb = wid * TPB
    pltpu.sync_copy(d0_hbm.at[pl.ds(cb, UCH)], d0_v)
    pltpu.sync_copy(d1_hbm.at[pl.ds(cb, UCH)], d1_v)
    pltpu.sync_copy(rp0_hbm.at[pl.ds(cb, UCH)], rp0_v)
    pltpu.sync_copy(rp1_hbm.at[pl.ds(cb, UCH)], rp1_v)
    for c in range(UCH):
        v0 = d0_v[c, :]
        v1 = d1_v[c, :]
        g0 = jnp.minimum(jnp.maximum(v0 - (EC - 1), 0), 1)
        g1 = jnp.minimum(jnp.maximum(v1 - (EC - 1), 0), 1)
        d0s_v[c, :] = v0 - g0 * v0
        d1s_v[c, :] = v1 - g1 * v1
    b0 = (buf0a, buf0b)
    b1 = (buf1a, buf1b)
    ob = (obufa, obufb)
    cp0 = [None] * UCH
    cp1 = [None] * UCH
    ost = [None, None]
    for c in range(min(2, UCH)):
        cp0[c] = pltpu.async_copy(po_hbm.at[d0s_v.at[c]], b0[c % 2], g0s)
        cp1[c] = pltpu.async_copy(po_hbm.at[d1s_v.at[c]], b1[c % 2], g1s)
    for c in range(UCH):
        k = c % 2
        cp0[c].wait()
        cp1[c].wait()
        if ost[k] is not None:
            ost[k].wait()

        def rstep(r, acc):
            lm = 1 - jnp.minimum(
                jnp.abs(lax.iota(jnp.int32, L) - r), 1)
            lmf = lm.astype(jnp.float32)
            raw0 = jnp.sum(d0_v[c, :] * lm)
            raw1 = jnp.sum(d1_v[c, :] * lm)
            ge0 = jnp.minimum(jnp.maximum(raw0 - (EC - 1), 0), 1)
            ge1 = jnp.minimum(jnp.maximum(raw1 - (EC - 1), 0), 1)
            s0 = jnp.sum(rp0_v[c, :] * lmf) * (1 - ge0).astype(jnp.float32)
            s1 = jnp.sum(rp1_v[c, :] * lmf) * (1 - ge1).astype(jnp.float32)
            s0v = jnp.full((L,), s0, jnp.float32)
            s1v = jnp.full((L,), s1, jnp.float32)
            for kk in range(D // L):
                sl = pl.ds(kk * L, L)
                ob[k][r, sl] = s0v * b0[k][r, sl] + s1v * b1[k][r, sl]
            return acc

        lax.fori_loop(0, UW, rstep, 0)
        if c + 2 < UCH:
            cp0[c + 2] = pltpu.async_copy(po_hbm.at[d0s_v.at[c + 2]],
                                          b0[k], g0s)
            cp1[c + 2] = pltpu.async_copy(po_hbm.at[d1s_v.at[c + 2]],
                                          b1[k], g1s)
        ost[k] = pltpu.async_copy(ob[k], out_hbm.at[pl.ds(tb + c * UW, UW)],
                                  oss)
    for k in range(2):
        if ost[k] is not None:
            ost[k].wait()


def _unpack(po, d0r, d1r, rp0r, rp1r):
    mesh = plsc.VectorSubcoreMesh(core_axis_name="c", subcore_axis_name="s")
    f = pl.kernel(
        _unpack_body,
        out_type=jax.ShapeDtypeStruct((T, D), jnp.float32),
        mesh=mesh,
        compiler_params=pltpu.CompilerParams(needs_layout_passes=False),
        scratch_types=[
            pltpu.VMEM((UCH, UW), jnp.int32),
            pltpu.VMEM((UCH, UW), jnp.int32),
            pltpu.VMEM((UCH, UW), jnp.int32),
            pltpu.VMEM((UCH, UW), jnp.int32),
            pltpu.VMEM((UCH, UW), jnp.float32),
            pltpu.VMEM((UCH, UW), jnp.float32),
            pltpu.VMEM((UW, D), jnp.float32),
            pltpu.VMEM((UW, D), jnp.float32),
            pltpu.VMEM((UW, D), jnp.float32),
            pltpu.VMEM((UW, D), jnp.float32),
            pltpu.VMEM((UW, D), jnp.float32),
            pltpu.VMEM((UW, D), jnp.float32),
            pltpu.SemaphoreType.DMA,
            pltpu.SemaphoreType.DMA,
            pltpu.SemaphoreType.DMA,
        ],
    )
    return f(po, d0r, d1r, rp0r, rp1r)


def _attn_body(cnt_ref, px_ref, tok_ref, wq_ref, wk_ref, wv_ref, wo_ref,
               out_ref):
    cnt = cnt_ref[0, 0, 0]
    x = px_ref[...]
    ri = jax.lax.broadcasted_iota(jnp.int32, (CAP, 1), 0)
    act = (ri < cnt).astype(jnp.float32)
    xm = jnp.where(ri < cnt, x, 0.0)
    q = jnp.dot(xm, wq_ref[0],
                preferred_element_type=jnp.float32) * (1.0 / math.sqrt(DB))
    k = jnp.dot(xm, wk_ref[0], preferred_element_type=jnp.float32)
    v = jnp.dot(xm, wv_ref[0], preferred_element_type=jnp.float32)
    scores = jax.lax.dot_general(
        q, k, (((1,), (1,)), ((), ())),
        preferred_element_type=jnp.float32)
    tok = tok_ref[0]
    pos_k = tok % N
    b_k = tok // N
    pos_q = jnp.reshape(pos_k, (CAP, 1))
    b_q = jnp.reshape(b_k, (CAP, 1))
    ci = jax.lax.broadcasted_iota(jnp.int32, (1, CAP), 1)
    mask = (ci < cnt) & (b_q == b_k) & (pos_k <= pos_q)
    scores = jnp.where(mask, scores, -1e9)
    m = jnp.max(scores, axis=1, keepdims=True)
    p = jnp.exp(scores - m)
    ctx = jnp.dot(p, v, preferred_element_type=jnp.float32)
    ctx = ctx / jnp.sum(p, axis=1, keepdims=True)
    out_ref[...] = jnp.dot(ctx, wo_ref[0], preferred_element_type=jnp.float32) * act


def _attention(counts, packed_x, slot_tok, Wq, Wk, Wv, Wo):
    grid = (E,)
    return pl.pallas_call(
        _attn_body,
        grid=grid,
        in_specs=[
            pl.BlockSpec((1, 1, 1), lambda e: (e, 0, 0)),
            pl.BlockSpec((CAP, D), lambda e: (e, 0)),
            pl.BlockSpec((1, 1, CAP), lambda e: (e, 0, 0)),
            pl.BlockSpec((1, D, DB), lambda e: (e, 0, 0)),
            pl.BlockSpec((1, D, DB), lambda e: (e, 0, 0)),
            pl.BlockSpec((1, D, DB), lambda e: (e, 0, 0)),
            pl.BlockSpec((1, DB, D), lambda e: (e, 0, 0)),
        ],
        out_specs=pl.BlockSpec((CAP, D), lambda e: (e, 0)),
        out_shape=jax.ShapeDtypeStruct((EC, D), jnp.float32),
    )(counts, packed_x, slot_tok, Wq, Wk, Wv, Wo)


def kernel(hidden_states, position_ids, active_mask, W_router, Wq, Wk, Wv, Wo):
    x = hidden_states.reshape(T, D)
    wr_pad = jnp.pad(W_router, ((0, 0), (0, EP - E)))
    idx, rp, stats, e0c, e1c, rp0c, rp1c = _router(x, wr_pad)
    loss = stats[0, 0]
    maxvio = stats[0, 1]
    d0, d1, tok2d, packed = _sortpack(e0c.reshape(T), e1c.reshape(T), x)
    counts = stats[0, 2:2 + E].astype(jnp.int32)

    slot_tok = tok2d[:EC, 0].reshape(E, 1, CAP)
    po = _attention(counts.reshape(E, 1, 1), packed,
                    slot_tok, Wq, Wk, Wv, Wo)

    rp0r = rp0c.reshape(NW * UCH, UW)
    rp1r = rp1c.reshape(NW * UCH, UW)
    final = _unpack(po, d0.reshape(NW * UCH, UW), d1.reshape(NW * UCH, UW),
                    rp0r, rp1r).reshape(B, N, D)
    return final, loss, maxvio


# speedup vs baseline: 1.2621x; 1.0199x over previous
import functools
import math

import jax
import jax.numpy as jnp
import numpy as np
from jax import lax
from jax.experimental import pallas as pl
from jax.experimental.pallas import tpu as pltpu
from jax.experimental.pallas import tpu_sc as plsc

B, N, D = 2, 2048, 1024
E, TOPK, DB = 8, 2, 128
CAP = 1280
T = B * N
TK = T * TOPK
EP = 128
NEG = -1e30

L = 16
NS = 16
EC = E * CAP
PAD_ROWS = EC + L
MARK = EC + (1 << 20)
TPS = T // NS


def _router_body(x_ref, wr_ref, idx_ref, rp_ref, stats_ref):
    x = x_ref[...]
    wr = wr_ref[...]
    logits = jnp.dot(x, wr, preferred_element_type=jnp.float32)
    cols = jax.lax.broadcasted_iota(jnp.int32, (T, EP), 1)
    valid = cols < E
    logits = jnp.where(valid, logits, NEG)
    m = jnp.max(logits, axis=1, keepdims=True)
    p = jnp.exp(logits - m)
    probs = p / jnp.sum(p, axis=1, keepdims=True)
    m1 = jnp.max(probs, axis=1, keepdims=True)
    i1 = jnp.min(jnp.where(probs == m1, cols, EP), axis=1, keepdims=True)
    probs2 = jnp.where(cols == i1, -1.0, probs)
    m2 = jnp.max(probs2, axis=1, keepdims=True)
    i2 = jnp.min(jnp.where(probs2 == m2, cols, EP), axis=1, keepdims=True)
    ssum = m1 + m2 + 1e-9
    rp1 = m1 / ssum
    rp2 = m2 / ssum
    idx_ref[...] = jnp.concatenate([i1, i2], axis=1)
    rp_ref[...] = jnp.concatenate([rp1, rp2], axis=1)
    onehot = (cols == i1).astype(jnp.float32) + (cols == i2).astype(jnp.float32)
    counts = jnp.sum(onehot, axis=0, keepdims=True)
    psum = jnp.sum(probs, axis=0, keepdims=True)
    denom = float(T) + 1e-9
    f_e = counts / (TOPK * denom)
    p_e = psum / denom
    loss = E * jnp.sum(f_e * p_e, axis=1, keepdims=True)
    maxvio = jnp.max(f_e, axis=1, keepdims=True) * E - 1.0
    stats_ref[...] = jnp.concatenate(
        [jnp.concatenate([loss, maxvio], axis=1), counts[:, : EP - 2]], axis=1)


def _router(x, wr_pad):
    return pl.pallas_call(
        _router_body,
        out_shape=(
            jax.ShapeDtypeStruct((T, 2), jnp.int32),
            jax.ShapeDtypeStruct((T, 2), jnp.float32),
            jax.ShapeDtypeStruct((1, EP), jnp.float32),
        ),
    )(x, wr_pad)


NW = 32
CW = 32
TPB = T // NW
CH = TPB // CW
TKW = 128


def _sortpack_body(e0_hbm, e1_hbm, x_hbm,
                   d0_hbm, d1_hbm, tok2d_hbm, packed_hbm,
                   e0_v, e1_v, p0_v, p1_v,
                   tmp_v, comb0_v, comb1_v, e0r_v, e1r_v,
                   d0o_v, d1o_v, d0c_v, d1c_v, rows_tok, xbuf0, xbuf1,
                   sh_p0, sh_p1, sem, sem2, sem3):
    cid = lax.axis_index("c")
    sid = lax.axis_index("s")
    wid = sid * 2 + cid

    def eq1(v, s):
        return 1 - jnp.minimum(jnp.abs(v - s), 1)

    @pl.when(sid < E)
    def _scan():
        e = sid
        pltpu.sync_copy(e0_hbm, e0_v)
        pltpu.sync_copy(e1_hbm, e1_v)
        esplat = jnp.full((L,), e, jnp.int32)

        def step(i, carry):
            sl = pl.ds(i * L, L)
            m0 = eq1(e0_v[sl], esplat)
            m1 = eq1(e1_v[sl], esplat)
            c0 = plsc.cumsum(m0)
            c1 = plsc.cumsum(m1)
            i1 = c1 - m1
            pos0 = carry + (c0 - m0) + i1
            pos1 = carry + c0 + i1
            p0_v[sl] = m0 * (pos0 + 1)
            p1_v[sl] = m1 * (pos1 + 1)
            s = jnp.sum(m0) + jnp.sum(m1)
            return carry + s

        lax.fori_loop(0, T // L, step, jnp.zeros((L,), jnp.int32))
        pltpu.sync_copy(p0_v, sh_p0.at[e])
        pltpu.sync_copy(p1_v, sh_p1.at[e])

    plsc.subcore_barrier()

    tb = wid * TPB
    pltpu.sync_copy(e0_hbm.at[pl.ds(tb, TPB)], e0r_v)
    pltpu.sync_copy(e1_hbm.at[pl.ds(tb, TPB)], e1r_v)
    pltpu.sync_copy(sh_p0.at[0, pl.ds(tb, TPB)], comb0_v)
    pltpu.sync_copy(sh_p1.at[0, pl.ds(tb, TPB)], comb1_v)
    for e in range(1, E):
        pltpu.sync_copy(sh_p0.at[e, pl.ds(tb, TPB)], tmp_v)
        for j in range(TPB // L):
            sl = pl.ds(j * L, L)
            comb0_v[sl] = comb0_v[sl] + tmp_v[sl]
        pltpu.sync_copy(sh_p1.at[e, pl.ds(tb, TPB)], tmp_v)
        for j in range(TPB // L):
            sl = pl.ds(j * L, L)
            comb1_v[sl] = comb1_v[sl] + tmp_v[sl]
    for j in range(TPB // L):
        sl = pl.ds(j * L, L)
        p0 = comb0_v[sl] - 1
        p1 = comb1_v[sl] - 1
        ge0 = jnp.minimum(jnp.maximum(p0 - (CAP - 1), 0), 1)
        ge1 = jnp.minimum(jnp.maximum(p1 - (CAP - 1), 0), 1)
        dd0 = e0r_v[sl] * CAP + p0
        dd1 = e1r_v[sl] * CAP + p1
        d0o_v[sl] = dd0 + ge0 * (MARK - dd0)
        d1o_v[sl] = dd1 + ge1 * (MARK - dd1)
        csl = pl.ds((j % (CW // L)) * L, L)
        d0c_v[j // (CW // L), csl] = dd0 + ge0 * (EC - dd0)
        d1c_v[j // (CW // L), csl] = dd1 + ge1 * (EC - dd1)
    pltpu.sync_copy(d0o_v, d0_hbm.at[pl.ds(tb, TPB)])
    pltpu.sync_copy(d1o_v, d1_hbm.at[pl.ds(tb, TPB)])

    bufs = (xbuf0, xbuf1)
    lds = [None] * CH
    for c in range(min(2, CH)):
        lds[c] = pltpu.async_copy(
            x_hbm.at[pl.ds(tb + c * CW, CW)], bufs[c % 2], sem3)
    for c in range(CH):
        cbase = tb + c * CW

        def rstep(j, acc):
            for kk in range(TKW // L):
                rows_tok[j, pl.ds(kk * L, L)] = jnp.full((L,), cbase + j,
                                                         jnp.int32)
            return acc

        lax.fori_loop(0, CW, rstep, 0)
        t0 = pltpu.async_copy(rows_tok, tok2d_hbm.at[d0c_v.at[c]], sem)
        t1 = pltpu.async_copy(rows_tok, tok2d_hbm.at[d1c_v.at[c]], sem2)
        lds[c].wait()
        s0 = pltpu.async_copy(bufs[c % 2], packed_hbm.at[d0c_v.at[c]], sem)
        s1 = pltpu.async_copy(bufs[c % 2], packed_hbm.at[d1c_v.at[c]], sem2)
        t0.wait()
        t1.wait()
        s0.wait()
        s1.wait()
        if c + 2 < CH:
            lds[c + 2] = pltpu.async_copy(
                x_hbm.at[pl.ds(tb + (c + 2) * CW, CW)], bufs[c % 2], sem3)


def _sortpack(e0, e1, x):
    mesh = plsc.VectorSubcoreMesh(core_axis_name="c", subcore_axis_name="s")
    f = pl.kernel(
        _sortpack_body,
        out_type=(
            jax.ShapeDtypeStruct((T,), jnp.int32),
            jax.ShapeDtypeStruct((T,), jnp.int32),
            jax.ShapeDtypeStruct((PAD_ROWS, TKW), jnp.int32),
            jax.ShapeDtypeStruct((PAD_ROWS, D), jnp.float32),
        ),
        mesh=mesh,
        compiler_params=pltpu.CompilerParams(needs_layout_passes=False),
        scratch_types=[
            pltpu.VMEM((T,), jnp.int32),
            pltpu.VMEM((T,), jnp.int32),
            pltpu.VMEM((T,), jnp.int32),
            pltpu.VMEM((T,), jnp.int32),
            pltpu.VMEM((TPB,), jnp.int32),
            pltpu.VMEM((TPB,), jnp.int32),
            pltpu.VMEM((TPB,), jnp.int32),
            pltpu.VMEM((TPB,), jnp.int32),
            pltpu.VMEM((TPB,), jnp.int32),
            pltpu.VMEM((TPB,), jnp.int32),
            pltpu.VMEM((TPB,), jnp.int32),
            pltpu.VMEM((CH, CW), jnp.int32),
            pltpu.VMEM((CH, CW), jnp.int32),
            pltpu.VMEM((CW, TKW), jnp.int32),
            pltpu.VMEM((CW, D), jnp.float32),
            pltpu.VMEM((CW, D), jnp.float32),
            pltpu.VMEM_SHARED((E, T), jnp.int32),
            pltpu.VMEM_SHARED((E, T), jnp.int32),
            pltpu.SemaphoreType.DMA,
            pltpu.SemaphoreType.DMA,
            pltpu.SemaphoreType.DMA,
        ],
    )
    return f(e0, e1, x)


UW = 16
UCH = TPB // UW


def _unpack_body(po_hbm, d0_hbm, d1_hbm, rp0_hbm, rp1_hbm, out_hbm,
                 d0_v, d1_v, d0s_v, d1s_v, rp0_v, rp1_v,
                 buf0a, buf0b, buf1a, buf1b, obufa, obufb,
                 g0s, g1s, oss):
    wid = lax.axis_index("s") * 2 + lax.axis_index("c")
    cb = wid * UCH
    tb = wid * TPB
    pltpu.sync_copy(d0_hbm.at[pl.ds(cb, UCH)], d0_v)
    pltpu.sync_copy(d1_hbm.at[pl.ds(cb, UCH)], d1_v)
    pltpu.sync_copy(rp0_hbm.at[pl.ds(cb, UCH)], rp0_v)
    pltpu.sync_copy(rp1_hbm.at[pl.ds(cb, UCH)], rp1_v)
    for c in range(UCH):
        v0 = d0_v[c, :]
        v1 = d1_v[c, :]
        g0 = jnp.minimum(jnp.maximum(v0 - (EC - 1), 0), 1)
        g1 = jnp.minimum(jnp.maximum(v1 - (EC - 1), 0), 1)
        d0s_v[c, :] = v0 - g0 * v0
        d1s_v[c, :] = v1 - g1 * v1
    b0 = (buf0a, buf0b)
    b1 = (buf1a, buf1b)
    ob = (obufa, obufb)
    cp0 = [None] * UCH
    cp1 = [None] * UCH
    ost = [None, None]
    for c in range(min(2, UCH)):
        cp0[c] = pltpu.async_copy(po_hbm.at[d0s_v.at[c]], b0[c % 2], g0s)
        cp1[c] = pltpu.async_copy(po_hbm.at[d1s_v.at[c]], b1[c % 2], g1s)
    for c in range(UCH):
        k = c % 2
        cp0[c].wait()
        cp1[c].wait()
        if ost[k] is not None:
            ost[k].wait()

        def rstep(r, acc):
            lm = 1 - jnp.minimum(
                jnp.abs(lax.iota(jnp.int32, L) - r), 1)
            lmf = lm.astype(jnp.float32)
            raw0 = jnp.sum(d0_v[c, :] * lm)
            raw1 = jnp.sum(d1_v[c, :] * lm)
            ge0 = jnp.minimum(jnp.maximum(raw0 - (EC - 1), 0), 1)
            ge1 = jnp.minimum(jnp.maximum(raw1 - (EC - 1), 0), 1)
            s0 = jnp.sum(rp0_v[c, :] * lmf) * (1 - ge0).astype(jnp.float32)
            s1 = jnp.sum(rp1_v[c, :] * lmf) * (1 - ge1).astype(jnp.float32)
            s0v = jnp.full((L,), s0, jnp.float32)
            s1v = jnp.full((L,), s1, jnp.float32)
            for kk in range(D // L):
                sl = pl.ds(kk * L, L)
                ob[k][r, sl] = s0v * b0[k][r, sl] + s1v * b1[k][r, sl]
            return acc

        lax.fori_loop(0, UW, rstep, 0)
        if c + 2 < UCH:
            cp0[c + 2] = pltpu.async_copy(po_hbm.at[d0s_v.at[c + 2]],
                                          b0[k], g0s)
            cp1[c + 2] = pltpu.async_copy(po_hbm.at[d1s_v.at[c + 2]],
                                          b1[k], g1s)
        ost[k] = pltpu.async_copy(ob[k], out_hbm.at[pl.ds(tb + c * UW, UW)],
                                  oss)
    for k in range(2):
        if ost[k] is not None:
            ost[k].wait()


def _unpack(po, d0r, d1r, rp0r, rp1r):
    mesh = plsc.VectorSubcoreMesh(core_axis_name="c", subcore_axis_name="s")
    f = pl.kernel(
        _unpack_body,
        out_type=jax.ShapeDtypeStruct((T, D), jnp.float32),
        mesh=mesh,
        compiler_params=pltpu.CompilerParams(needs_layout_passes=False),
        scratch_types=[
            pltpu.VMEM((UCH, UW), jnp.int32),
            pltpu.VMEM((UCH, UW), jnp.int32),
            pltpu.VMEM((UCH, UW), jnp.int32),
            pltpu.VMEM((UCH, UW), jnp.int32),
            pltpu.VMEM((UCH, UW), jnp.float32),
            pltpu.VMEM((UCH, UW), jnp.float32),
            pltpu.VMEM((UW, D), jnp.float32),
            pltpu.VMEM((UW, D), jnp.float32),
            pltpu.VMEM((UW, D), jnp.float32),
            pltpu.VMEM((UW, D), jnp.float32),
            pltpu.VMEM((UW, D), jnp.float32),
            pltpu.VMEM((UW, D), jnp.float32),
            pltpu.SemaphoreType.DMA,
            pltpu.SemaphoreType.DMA,
            pltpu.SemaphoreType.DMA,
        ],
    )
    return f(po, d0r, d1r, rp0r, rp1r)


def _attn_body(cnt_ref, px_ref, tok_ref, wq_ref, wk_ref, wv_ref, wo_ref,
               out_ref):
    cnt = cnt_ref[0, 0, 0]
    x = px_ref[...]
    ri = jax.lax.broadcasted_iota(jnp.int32, (CAP, 1), 0)
    act = (ri < cnt).astype(jnp.float32)
    xm = jnp.where(ri < cnt, x, 0.0)
    q = jnp.dot(xm, wq_ref[0],
                preferred_element_type=jnp.float32) * (1.0 / math.sqrt(DB))
    k = jnp.dot(xm, wk_ref[0], preferred_element_type=jnp.float32)
    v = jnp.dot(xm, wv_ref[0], preferred_element_type=jnp.float32)
    scores = jax.lax.dot_general(
        q, k, (((1,), (1,)), ((), ())),
        preferred_element_type=jnp.float32)
    tok = tok_ref[0]
    pos_k = tok % N
    b_k = tok // N
    pos_q = jnp.reshape(pos_k, (CAP, 1))
    b_q = jnp.reshape(b_k, (CAP, 1))
    ci = jax.lax.broadcasted_iota(jnp.int32, (1, CAP), 1)
    mask = (ci < cnt) & (b_q == b_k) & (pos_k <= pos_q)
    scores = jnp.where(mask, scores, -1e9)
    m = jnp.max(scores, axis=1, keepdims=True)
    p = jnp.exp(scores - m)
    ctx = jnp.dot(p, v, preferred_element_type=jnp.float32)
    ctx = ctx / jnp.sum(p, axis=1, keepdims=True)
    out_ref[...] = jnp.dot(ctx, wo_ref[0], preferred_element_type=jnp.float32) * act


def _attention(counts, packed_x, slot_tok, Wq, Wk, Wv, Wo):
    grid = (E,)
    return pl.pallas_call(
        _attn_body,
        grid=grid,
        in_specs=[
            pl.BlockSpec((1, 1, 1), lambda e: (e, 0, 0)),
            pl.BlockSpec((CAP, D), lambda e: (e, 0)),
            pl.BlockSpec((1, 1, CAP), lambda e: (e, 0, 0)),
            pl.BlockSpec((1, D, DB), lambda e: (e, 0, 0)),
            pl.BlockSpec((1, D, DB), lambda e: (e, 0, 0)),
            pl.BlockSpec((1, D, DB), lambda e: (e, 0, 0)),
            pl.BlockSpec((1, DB, D), lambda e: (e, 0, 0)),
        ],
        out_specs=pl.BlockSpec((CAP, D), lambda e: (e, 0)),
        out_shape=jax.ShapeDtypeStruct((EC, D), jnp.float32),
    )(counts, packed_x, slot_tok, Wq, Wk, Wv, Wo)


def kernel(hidden_states, position_ids, active_mask, W_router, Wq, Wk, Wv, Wo):
    x = hidden_states.reshape(T, D)
    wr_pad = jnp.pad(W_router, ((0, 0), (0, EP - E)))
    idx, rp, stats = _router(x, wr_pad)
    loss = stats[0, 0]
    maxvio = stats[0, 1]
    d0, d1, tok2d, packed = _sortpack(idx[:, 0], idx[:, 1], x)
    counts = stats[0, 2:2 + E].astype(jnp.int32)

    slot_tok = tok2d[:EC, 0].reshape(E, 1, CAP)
    po = _attention(counts.reshape(E, 1, 1), packed,
                    slot_tok, Wq, Wk, Wv, Wo)

    rp0r = rp[:, 0].reshape(NW * UCH, UW)
    rp1r = rp[:, 1].reshape(NW * UCH, UW)
    final = _unpack(po, d0.reshape(NW * UCH, UW), d1.reshape(NW * UCH, UW),
                    rp0r, rp1r).reshape(B, N, D)
    return final, loss, maxvio
